# baseline (speedup 1.0000x reference)
import functools

import jax
import jax.numpy as jnp
from jax.experimental import pallas as pl
from jax.experimental.pallas import tpu as pltpu

_EPS2 = 1e-6
_NB = 512
_IMAX = 2**31 - 1


def _argmin_body(p8m2_ref, mt_ref, u_ref, bestd_ref, besti_ref, umin_ref):
    i = pl.program_id(0)
    nb = mt_ref.shape[1]
    b = p8m2_ref.shape[0]
    mt = mt_ref[...]
    s = jnp.dot(p8m2_ref[...], mt, preferred_element_type=jnp.float32)
    msq = jnp.sum(mt * mt, axis=0, keepdims=True)
    d2 = s + msq
    rowmin = jnp.min(d2, axis=1, keepdims=True)
    gcol = jax.lax.broadcasted_iota(jnp.int32, (b, nb), 1) + i * nb
    cand = jnp.min(jnp.where(d2 == rowmin, gcol, _IMAX), axis=1, keepdims=True)
    ulocal = jnp.min(u_ref[0])

    @pl.when(i == 0)
    def _():
        bestd_ref[...] = rowmin
        besti_ref[...] = cand
        umin_ref[...] = jnp.full(umin_ref.shape, ulocal, jnp.int32)

    @pl.when(i > 0)
    def _():
        prev = bestd_ref[...]
        better = rowmin < prev
        bestd_ref[...] = jnp.where(better, rowmin, prev)
        besti_ref[...] = jnp.where(better, cand, besti_ref[...])
        umin_ref[...] = jnp.minimum(umin_ref[...],
                                    jnp.full(umin_ref.shape, ulocal, jnp.int32))


def _compact_body(u_ref, umin_ref, free_ref, c_ref):
    i = pl.program_id(0)
    nb = u_ref.shape[2]
    b = free_ref.shape[0]
    c0 = jnp.where(i == 0, 0, c_ref[0])
    m = u_ref[0] == jnp.min(umin_ref[...])
    mf = m.astype(jnp.float32)
    tri = (jax.lax.broadcasted_iota(jnp.int32, (nb, nb), 0)
           <= jax.lax.broadcasted_iota(jnp.int32, (nb, nb), 1)).astype(jnp.float32)
    pos = jnp.dot(mf, tri, preferred_element_type=jnp.float32,
                  precision=jax.lax.Precision.HIGHEST)
    pos = pos + c0.astype(jnp.float32)
    rio = jax.lax.broadcasted_iota(jnp.int32, (b, nb), 0).astype(jnp.float32)
    a = jnp.where((rio + 1.0 == jnp.broadcast_to(pos, (b, nb)))
                  & jnp.broadcast_to(m, (b, nb)), 1.0, 0.0)
    gj8 = (jax.lax.broadcasted_iota(jnp.int32, (nb, 8), 0) + i * nb).astype(jnp.float32)
    contrib = jnp.dot(a, gj8, preferred_element_type=jnp.float32,
                      precision=jax.lax.Precision.HIGHEST)

    @pl.when(i == 0)
    def _():
        free_ref[...] = contrib

    @pl.when(i > 0)
    def _():
        free_ref[...] = free_ref[...] + contrib

    c_ref[0] = c0 + jnp.sum(m.astype(jnp.int32))


def _address_body(bestd_ref, besti_ref, free_ref, p8m2_ref, desc_ref, out_ref):
    b = bestd_ref.shape[0]
    f32 = jnp.float32
    eye = (jax.lax.broadcasted_iota(jnp.int32, (b, b), 0)
           == jax.lax.broadcasted_iota(jnp.int32, (b, b), 1)).astype(f32)
    iot0 = jax.lax.broadcasted_iota(jnp.int32, (b, b), 0).astype(f32)
    iot1 = jax.lax.broadcasted_iota(jnp.int32, (b, b), 1).astype(f32)

    p8m2 = p8m2_ref[...]
    psq = jnp.sum(p8m2 * p8m2, axis=1, keepdims=True) * 0.25
    d2 = bestd_ref[...] + psq
    mask = d2 > _EPS2
    mf = mask.astype(f32)
    ltri = (iot1 <= iot0)
    cum = jnp.dot(ltri.astype(f32), mf, preferred_element_type=f32,
                  precision=jax.lax.Precision.HIGHEST)
    rank = jnp.clip(cum - 1.0, 0.0, float(b - 1))
    o1 = (iot1 == jnp.broadcast_to(rank, (b, b))).astype(f32)
    fsel8 = jnp.dot(o1, free_ref[...], preferred_element_type=f32,
                    precision=jax.lax.Precision.HIGHEST)
    idx = jnp.where(mask, fsel8[:, 0:1], besti_ref[...].astype(f32))
    idx_row = jnp.sum(eye * jnp.broadcast_to(idx, (b, b)), axis=0, keepdims=True)
    e = jnp.broadcast_to(idx, (b, b)) == jnp.broadcast_to(idx_row, (b, b))
    lw_row = jnp.max(jnp.where(e, iot0, -1.0), axis=0, keepdims=True)
    lw_col = jnp.sum(eye * jnp.broadcast_to(lw_row, (b, b)), axis=1, keepdims=True)
    g = (jnp.broadcast_to(lw_col, (b, b)) == iot1).astype(f32)
    out_ref[...] = jnp.dot(g, desc_ref[...], preferred_element_type=f32,
                           precision=jax.lax.Precision.HIGHEST)


@jax.jit
def kernel(points, descriptors, mem_points, mem_descriptors, usage):
    del mem_descriptors
    b = points.shape[0]
    n = mem_points.shape[0]
    f = descriptors.shape[1]
    g = (n + _NB - 1) // _NB
    npad = g * _NB

    mt = jnp.full((8, npad), 0.0, jnp.float32)
    mt = mt.at[:3, :n].set(mem_points.T).at[:3, n:].set(1e18)
    p8m2 = jnp.zeros((b, 8), jnp.float32).at[:, :3].set(points * -2.0)
    u_r = jnp.full((npad,), _IMAX, jnp.int32).at[:n].set(usage).reshape(g, 1, _NB)

    bestd, besti, umin8 = pl.pallas_call(
        _argmin_body,
        grid=(g,),
        in_specs=[
            pl.BlockSpec((b, 8), lambda i: (0, 0)),
            pl.BlockSpec((8, _NB), lambda i: (0, i)),
            pl.BlockSpec((1, 1, _NB), lambda i: (i, 0, 0)),
        ],
        out_specs=[
            pl.BlockSpec((b, 1), lambda i: (0, 0)),
            pl.BlockSpec((b, 1), lambda i: (0, 0)),
            pl.BlockSpec((8, 128), lambda i: (0, 0)),
        ],
        out_shape=[
            jax.ShapeDtypeStruct((b, 1), jnp.float32),
            jax.ShapeDtypeStruct((b, 1), jnp.int32),
            jax.ShapeDtypeStruct((8, 128), jnp.int32),
        ],
    )(p8m2, mt, u_r)

    free8 = pl.pallas_call(
        _compact_body,
        grid=(g,),
        in_specs=[
            pl.BlockSpec((1, 1, _NB), lambda i: (i, 0, 0)),
            pl.BlockSpec((8, 128), lambda i: (0, 0)),
        ],
        out_specs=pl.BlockSpec((b, 8), lambda i: (0, 0)),
        out_shape=jax.ShapeDtypeStruct((b, 8), jnp.float32),
        scratch_shapes=[pltpu.SMEM((1,), jnp.int32)],
    )(u_r, umin8)

    out = pl.pallas_call(
        _address_body,
        in_specs=[pl.BlockSpec(x.shape, lambda: (0,) * x.ndim)
                  for x in (bestd, besti, free8, p8m2, descriptors)],
        out_specs=pl.BlockSpec((b, f), lambda: (0, 0)),
        out_shape=jax.ShapeDtypeStruct((b, f), jnp.float32),
    )(bestd, besti, free8, p8m2, descriptors)
    return out


# speedup vs baseline: 2.5707x; 2.5707x over previous
import functools

import jax
import jax.numpy as jnp
from jax.experimental import pallas as pl
from jax.experimental.pallas import tpu as pltpu

_EPS2 = 1e-6
_NB = 512
_IMAX = 2**31 - 1


def _argmin_body(p8m2_ref, mt_ref, u_ref, iota_ref, bestd_ref, besti_ref,
                 umin_ref):
    i = pl.program_id(0)
    nb = mt_ref.shape[1]
    b = p8m2_ref.shape[0]
    mt = mt_ref[...]
    s = jnp.dot(p8m2_ref[...], mt, preferred_element_type=jnp.float32)
    msq = jnp.sum(mt * mt, axis=0, keepdims=True)
    d2 = s + msq
    rowmin = jnp.min(d2, axis=1, keepdims=True)
    cand = jnp.min(jnp.where(d2 == rowmin, iota_ref[...], _IMAX), axis=1,
                   keepdims=True) + i * nb
    ulocal = jnp.min(u_ref[0])

    @pl.when(i == 0)
    def _():
        bestd_ref[...] = rowmin
        besti_ref[...] = cand
        umin_ref[...] = jnp.full(umin_ref.shape, ulocal, jnp.int32)

    @pl.when(i > 0)
    def _():
        prev = bestd_ref[...]
        better = rowmin < prev
        bestd_ref[...] = jnp.where(better, rowmin, prev)
        besti_ref[...] = jnp.where(better, cand, besti_ref[...])
        umin_ref[...] = jnp.minimum(umin_ref[...],
                                    jnp.full(umin_ref.shape, ulocal, jnp.int32))


def _compact_body(u_ref, umin_ref, free_ref, c_ref):
    i = pl.program_id(0)
    nb = u_ref.shape[2]
    b = free_ref.shape[0]
    c0 = jnp.where(i == 0, 0, c_ref[0])

    @pl.when(c0 < b)
    def _():
        m = u_ref[0] == jnp.min(umin_ref[...])
        mf = m.astype(jnp.float32)
        tri = (jax.lax.broadcasted_iota(jnp.int32, (nb, nb), 0)
               <= jax.lax.broadcasted_iota(jnp.int32, (nb, nb), 1)
               ).astype(jnp.float32)
        pos = jnp.dot(mf, tri, preferred_element_type=jnp.float32)
        pos = pos + c0.astype(jnp.float32)
        rio = jax.lax.broadcasted_iota(jnp.int32, (b, nb), 0).astype(jnp.float32)
        a = jnp.where((rio + 1.0 == jnp.broadcast_to(pos, (b, nb)))
                      & jnp.broadcast_to(m, (b, nb)), 1.0, 0.0)
        gj8 = (jax.lax.broadcasted_iota(jnp.int32, (nb, 8), 0)
               + i * nb).astype(jnp.float32)
        contrib = jnp.dot(a, gj8, preferred_element_type=jnp.float32,
                          precision=jax.lax.Precision.HIGHEST)

        @pl.when(i == 0)
        def _():
            free_ref[...] = contrib

        @pl.when(i > 0)
        def _():
            free_ref[...] = free_ref[...] + contrib

        c_ref[0] = c0 + jnp.sum(m.astype(jnp.int32))


def _address_body(bestd_ref, besti_ref, free_ref, p8m2_ref, desc_ref, out_ref):
    b = bestd_ref.shape[0]
    f32 = jnp.float32
    eye = (jax.lax.broadcasted_iota(jnp.int32, (b, b), 0)
           == jax.lax.broadcasted_iota(jnp.int32, (b, b), 1)).astype(f32)
    iot0 = jax.lax.broadcasted_iota(jnp.int32, (b, b), 0).astype(f32)
    iot1 = jax.lax.broadcasted_iota(jnp.int32, (b, b), 1).astype(f32)

    p8m2 = p8m2_ref[...]
    psq = jnp.sum(p8m2 * p8m2, axis=1, keepdims=True) * 0.25
    d2 = bestd_ref[...] + psq
    mask = d2 > _EPS2
    mf = mask.astype(f32)
    ltri = (iot1 <= iot0)
    cum = jnp.dot(ltri.astype(f32), mf, preferred_element_type=f32)
    rank = jnp.clip(cum - 1.0, 0.0, float(b - 1))
    o1 = (iot1 == jnp.broadcast_to(rank, (b, b))).astype(f32)
    fsel8 = jnp.dot(o1, free_ref[...], preferred_element_type=f32,
                    precision=jax.lax.Precision.HIGHEST)
    idx = jnp.where(mask, fsel8[:, 0:1], besti_ref[...].astype(f32))
    idx_row = jnp.sum(eye * jnp.broadcast_to(idx, (b, b)), axis=0, keepdims=True)
    e = jnp.broadcast_to(idx, (b, b)) == jnp.broadcast_to(idx_row, (b, b))
    lw_row = jnp.max(jnp.where(e, iot0, -1.0), axis=0, keepdims=True)
    lw_col = jnp.sum(eye * jnp.broadcast_to(lw_row, (b, b)), axis=1, keepdims=True)
    g = (jnp.broadcast_to(lw_col, (b, b)) == iot1).astype(f32)
    out_ref[...] = jnp.dot(g, desc_ref[...], preferred_element_type=f32,
                           precision=jax.lax.Precision.HIGHEST)


@jax.jit
def kernel(points, descriptors, mem_points, mem_descriptors, usage):
    del mem_descriptors
    b = points.shape[0]
    n = mem_points.shape[0]
    f = descriptors.shape[1]
    g = (n + _NB - 1) // _NB
    npad = g * _NB

    mt = jnp.full((8, npad), 0.0, jnp.float32)
    mt = mt.at[:3, :n].set(mem_points.T).at[:3, n:].set(1e18)
    p8m2 = jnp.zeros((b, 8), jnp.float32).at[:, :3].set(points * -2.0)
    u_r = jnp.full((npad,), _IMAX, jnp.int32).at[:n].set(usage).reshape(g, 1, _NB)
    iota_c = jnp.broadcast_to(jnp.arange(_NB, dtype=jnp.int32)[None, :],
                              (b, _NB))

    bestd, besti, umin8 = pl.pallas_call(
        _argmin_body,
        grid=(g,),
        in_specs=[
            pl.BlockSpec((b, 8), lambda i: (0, 0)),
            pl.BlockSpec((8, _NB), lambda i: (0, i)),
            pl.BlockSpec((1, 1, _NB), lambda i: (i, 0, 0)),
            pl.BlockSpec((b, _NB), lambda i: (0, 0)),
        ],
        out_specs=[
            pl.BlockSpec((b, 1), lambda i: (0, 0)),
            pl.BlockSpec((b, 1), lambda i: (0, 0)),
            pl.BlockSpec((8, 128), lambda i: (0, 0)),
        ],
        out_shape=[
            jax.ShapeDtypeStruct((b, 1), jnp.float32),
            jax.ShapeDtypeStruct((b, 1), jnp.int32),
            jax.ShapeDtypeStruct((8, 128), jnp.int32),
        ],
    )(p8m2, mt, u_r, iota_c)

    free8 = pl.pallas_call(
        _compact_body,
        grid=(g,),
        in_specs=[
            pl.BlockSpec((1, 1, _NB), lambda i: (i, 0, 0)),
            pl.BlockSpec((8, 128), lambda i: (0, 0)),
        ],
        out_specs=pl.BlockSpec((b, 8), lambda i: (0, 0)),
        out_shape=jax.ShapeDtypeStruct((b, 8), jnp.float32),
        scratch_shapes=[pltpu.SMEM((1,), jnp.int32)],
    )(u_r, umin8)

    out = pl.pallas_call(
        _address_body,
        in_specs=[pl.BlockSpec(x.shape, lambda: (0,) * x.ndim)
                  for x in (bestd, besti, free8, p8m2, descriptors)],
        out_specs=pl.BlockSpec((b, f), lambda: (0, 0)),
        out_shape=jax.ShapeDtypeStruct((b, f), jnp.float32),
    )(bestd, besti, free8, p8m2, descriptors)
    return out
